# baseline (speedup 1.0000x reference)
import jax
import jax.numpy as jnp
from jax.experimental import pallas as pl
from jax.experimental.pallas import tpu as pltpu

N_TOK = 32768
DIM = 128
NSEG = 16
TILE = 16384
GRID = N_TOK // TILE


def _fused_kernel(x_ref, lo_ref, hi_ref, w1_ref, b1_ref, w2_ref, b2_ref,
                  w3_ref, b3_ref, out_ref, acc_ref, cnt_ref):
    pid = pl.program_id(0)

    @pl.when(pid == 0)
    def _init():
        acc_ref[...] = jnp.zeros_like(acc_ref)
        cnt_ref[...] = jnp.zeros_like(cnt_ref)

    x = x_ref[...]
    h1 = jnp.dot(x, w1_ref[...], preferred_element_type=jnp.float32) + b1_ref[...]
    h1 = jnp.where(h1 >= 0, h1, 0.2 * h1)
    h2 = jnp.dot(h1, w2_ref[...], preferred_element_type=jnp.float32) + b2_ref[...]
    h2 = jnp.where(h2 >= 0, h2, 0.2 * h2)

    cols = jax.lax.broadcasted_iota(jnp.int32, (NSEG, TILE), 1) + pid * TILE
    m = jnp.logical_and(cols >= lo_ref[...], cols < hi_ref[...]).astype(jnp.float32)

    acc_ref[...] += jnp.dot(m, h2, preferred_element_type=jnp.float32)
    cnt_ref[...] += jnp.sum(m, axis=1, keepdims=True)

    @pl.when(pid == GRID - 1)
    def _finish():
        out_ref[...] = (
            jnp.dot(acc_ref[...], w3_ref[...], preferred_element_type=jnp.float32)
            + cnt_ref[...] * b3_ref[...]
        )


def kernel(h_dag, obs_ptr, W1, b1, W2, b2, W3, b3):
    lo = obs_ptr[:-1].astype(jnp.int32).reshape(NSEG, 1)
    hi = obs_ptr[1:].astype(jnp.int32).reshape(NSEG, 1)

    const = lambda *_: (0, 0)
    out = pl.pallas_call(
        _fused_kernel,
        grid=(GRID,),
        in_specs=[
            pl.BlockSpec((TILE, DIM), lambda i: (i, 0)),
            pl.BlockSpec((NSEG, 1), const),
            pl.BlockSpec((NSEG, 1), const),
            pl.BlockSpec((DIM, 32), const),
            pl.BlockSpec((1, 32), const),
            pl.BlockSpec((32, 16), const),
            pl.BlockSpec((1, 16), const),
            pl.BlockSpec((16, DIM), const),
            pl.BlockSpec((1, DIM), const),
        ],
        out_specs=pl.BlockSpec((NSEG, DIM), const),
        out_shape=jax.ShapeDtypeStruct((NSEG, DIM), jnp.float32),
        scratch_shapes=[
            pltpu.VMEM((NSEG, 16), jnp.float32),
            pltpu.VMEM((NSEG, 1), jnp.float32),
        ],
        compiler_params=pltpu.CompilerParams(
            dimension_semantics=("arbitrary",),
        ),
    )(h_dag, lo, hi, W1, b1.reshape(1, 32), W2, b2.reshape(1, 16),
      W3, b3.reshape(1, DIM))
    return out


# speedup vs baseline: 1.0214x; 1.0214x over previous
import jax
import jax.numpy as jnp
from jax.experimental import pallas as pl
from jax.experimental.pallas import tpu as pltpu

N_TOK = 32768
DIM = 128
NSEG = 16
TILE = 4096
GRID = 4


def _mlp_block(x, w1, b1, w2, b2):
    h1 = jnp.dot(x, w1, preferred_element_type=jnp.float32) + b1
    h1 = jnp.where(h1 >= 0, h1, 0.2 * h1)
    h2 = jnp.dot(h1, w2, preferred_element_type=jnp.float32) + b2
    return jnp.where(h2 >= 0, h2, 0.2 * h2)


def _seg_mask(start, lo, hi):
    cols = jax.lax.broadcasted_iota(jnp.int32, (NSEG, TILE), 1) + start
    return jnp.logical_and(cols >= lo, cols < hi).astype(jnp.float32)


def _fused_kernel(xa_ref, xb_ref, lo_ref, hi_ref, w1_ref, b1_ref, w2_ref,
                  b2_ref, w3_ref, b3_ref, out_ref, acc_ref):
    pid = pl.program_id(0)

    @pl.when(pid == 0)
    def _init():
        acc_ref[...] = jnp.zeros_like(acc_ref)

    w1, b1 = w1_ref[...], b1_ref[...]
    w2, b2 = w2_ref[...], b2_ref[...]
    lo, hi = lo_ref[...], hi_ref[...]

    h2a = _mlp_block(xa_ref[...], w1, b1, w2, b2)
    ma = _seg_mask(pid * TILE, lo, hi)
    h2b = _mlp_block(xb_ref[...], w1, b1, w2, b2)
    mb = _seg_mask((GRID + pid) * TILE, lo, hi)

    acc_ref[...] += (jnp.dot(ma, h2a, preferred_element_type=jnp.float32)
                     + jnp.dot(mb, h2b, preferred_element_type=jnp.float32))

    @pl.when(pid == GRID - 1)
    def _finish():
        cnt = (hi - lo).astype(jnp.float32)
        out_ref[...] = (
            jnp.dot(acc_ref[...], w3_ref[...], preferred_element_type=jnp.float32)
            + cnt * b3_ref[...]
        )


def kernel(h_dag, obs_ptr, W1, b1, W2, b2, W3, b3):
    lo = obs_ptr[:-1].astype(jnp.int32).reshape(NSEG, 1)
    hi = obs_ptr[1:].astype(jnp.int32).reshape(NSEG, 1)

    const = lambda *_: (0, 0)
    out = pl.pallas_call(
        _fused_kernel,
        grid=(GRID,),
        in_specs=[
            pl.BlockSpec((TILE, DIM), lambda i: (i, 0)),
            pl.BlockSpec((TILE, DIM), lambda i: (i + GRID, 0)),
            pl.BlockSpec((NSEG, 1), const),
            pl.BlockSpec((NSEG, 1), const),
            pl.BlockSpec((DIM, 32), const),
            pl.BlockSpec((1, 32), const),
            pl.BlockSpec((32, 16), const),
            pl.BlockSpec((1, 16), const),
            pl.BlockSpec((16, DIM), const),
            pl.BlockSpec((1, DIM), const),
        ],
        out_specs=pl.BlockSpec((NSEG, DIM), const),
        out_shape=jax.ShapeDtypeStruct((NSEG, DIM), jnp.float32),
        scratch_shapes=[
            pltpu.VMEM((NSEG, 16), jnp.float32),
        ],
        compiler_params=pltpu.CompilerParams(
            dimension_semantics=("arbitrary",),
        ),
    )(h_dag, h_dag, lo, hi, W1, b1.reshape(1, 32), W2, b2.reshape(1, 16),
      W3, b3.reshape(1, DIM))
    return out


# speedup vs baseline: 1.1761x; 1.1514x over previous
import jax
import jax.numpy as jnp
from jax.experimental import pallas as pl
from jax.experimental.pallas import tpu as pltpu

N_TOK = 32768
DIM = 128
NSEG = 16
TILE = 8192
GRID = N_TOK // TILE


def _leaky(x):
    return jnp.maximum(x, 0.2 * x)


def _fused_kernel(ptr_ref, x_ref, w1_ref, b1_ref, w2_ref, b2_ref,
                  w3_ref, b3_ref, out_ref, acc_ref, lo_ref, hi_ref):
    pid = pl.program_id(0)

    @pl.when(pid == 0)
    def _init():
        acc_ref[...] = jnp.zeros_like(acc_ref)
        sub = jax.lax.broadcasted_iota(jnp.int32, (NSEG, 1), 0)
        lo = jnp.zeros((NSEG, 1), jnp.int32)
        hi = jnp.zeros((NSEG, 1), jnp.int32)
        for s in range(NSEG):
            lo = jnp.where(sub == s, ptr_ref[s], lo)
            hi = jnp.where(sub == s, ptr_ref[s + 1], hi)
        lo_ref[...] = lo
        hi_ref[...] = hi

    h1 = _leaky(jnp.dot(x_ref[...], w1_ref[...],
                        preferred_element_type=jnp.float32) + b1_ref[...])
    h2 = _leaky(jnp.dot(h1, w2_ref[...],
                        preferred_element_type=jnp.float32) + b2_ref[...])

    cols = jax.lax.broadcasted_iota(jnp.int32, (NSEG, TILE), 1) + pid * TILE
    m = jnp.logical_and(cols >= lo_ref[...], cols < hi_ref[...])

    acc_ref[...] += jnp.dot(m.astype(jnp.bfloat16), h2.astype(jnp.bfloat16),
                            preferred_element_type=jnp.float32)

    @pl.when(pid == GRID - 1)
    def _finish():
        cnt = (hi_ref[...] - lo_ref[...]).astype(jnp.float32)
        out_ref[...] = (
            jnp.dot(acc_ref[...], w3_ref[...], preferred_element_type=jnp.float32)
            + cnt * b3_ref[...]
        )


def kernel(h_dag, obs_ptr, W1, b1, W2, b2, W3, b3):
    const = lambda i, ptr: (0, 0)
    grid_spec = pltpu.PrefetchScalarGridSpec(
        num_scalar_prefetch=1,
        grid=(GRID,),
        in_specs=[
            pl.BlockSpec((TILE, DIM), lambda i, ptr: (i, 0)),
            pl.BlockSpec((DIM, 32), const),
            pl.BlockSpec((1, 32), const),
            pl.BlockSpec((32, 16), const),
            pl.BlockSpec((1, 16), const),
            pl.BlockSpec((16, DIM), const),
            pl.BlockSpec((1, DIM), const),
        ],
        out_specs=pl.BlockSpec((NSEG, DIM), const),
        scratch_shapes=[
            pltpu.VMEM((NSEG, 16), jnp.float32),
            pltpu.VMEM((NSEG, 1), jnp.int32),
            pltpu.VMEM((NSEG, 1), jnp.int32),
        ],
    )
    out = pl.pallas_call(
        _fused_kernel,
        grid_spec=grid_spec,
        out_shape=jax.ShapeDtypeStruct((NSEG, DIM), jnp.float32),
        compiler_params=pltpu.CompilerParams(
            dimension_semantics=("arbitrary",),
        ),
    )(obs_ptr, h_dag, W1, b1.reshape(1, 32), W2, b2.reshape(1, 16),
      W3, b3.reshape(1, DIM))
    return out


# speedup vs baseline: 1.2928x; 1.0992x over previous
import jax
import jax.numpy as jnp
from jax.experimental import pallas as pl
from jax.experimental.pallas import tpu as pltpu

N_TOK = 32768
DIM = 128
NSEG = 16
TILE = 8192
GRID = N_TOK // TILE


def _leaky(x):
    return jnp.maximum(x, 0.2 * x)


def _smem_to_col(ref, n, offset=0):
    sub = jax.lax.broadcasted_iota(jnp.int32, (n, 1), 0)
    col = jnp.zeros((n, 1), ref.dtype)
    for s in range(n):
        col = jnp.where(sub == s, ref[s + offset], col)
    return col


def _fused_kernel(ptr_ref, b1_ref, b2_ref, x_ref, w1_ref, w2_ref,
                  w3_ref, b3_ref, out_ref, acc_ref, lo_ref, hi_ref,
                  b1c_ref, b2c_ref):
    pid = pl.program_id(0)

    @pl.when(pid == 0)
    def _init():
        acc_ref[...] = jnp.zeros_like(acc_ref)
        lo_ref[...] = _smem_to_col(ptr_ref, NSEG)
        hi_ref[...] = _smem_to_col(ptr_ref, NSEG, offset=1)
        b1c_ref[...] = _smem_to_col(b1_ref, 32)
        b2c_ref[...] = _smem_to_col(b2_ref, NSEG)

    h1t = _leaky(
        jax.lax.dot_general(w1_ref[...], x_ref[...],
                            (((0,), (1,)), ((), ())),
                            preferred_element_type=jnp.float32)
        + b1c_ref[...])
    h2t = _leaky(
        jax.lax.dot_general(w2_ref[...], h1t,
                            (((0,), (0,)), ((), ())),
                            preferred_element_type=jnp.float32)
        + b2c_ref[...])

    cols = jax.lax.broadcasted_iota(jnp.int32, (NSEG, TILE), 1) + pid * TILE
    m = jnp.logical_and(cols >= lo_ref[...], cols < hi_ref[...])

    acc_ref[...] += jax.lax.dot_general(
        m.astype(jnp.bfloat16), h2t.astype(jnp.bfloat16),
        (((1,), (1,)), ((), ())),
        preferred_element_type=jnp.float32)

    @pl.when(pid == GRID - 1)
    def _finish():
        cnt = (hi_ref[...] - lo_ref[...]).astype(jnp.float32)
        out_ref[...] = (
            jnp.dot(acc_ref[...], w3_ref[...], preferred_element_type=jnp.float32)
            + cnt * b3_ref[...]
        )


def kernel(h_dag, obs_ptr, W1, b1, W2, b2, W3, b3):
    const = lambda i, *refs: (0, 0)
    grid_spec = pltpu.PrefetchScalarGridSpec(
        num_scalar_prefetch=3,
        grid=(GRID,),
        in_specs=[
            pl.BlockSpec((TILE, DIM), lambda i, *refs: (i, 0)),
            pl.BlockSpec((DIM, 32), const),
            pl.BlockSpec((32, 16), const),
            pl.BlockSpec((16, DIM), const),
            pl.BlockSpec((1, DIM), const),
        ],
        out_specs=pl.BlockSpec((NSEG, DIM), const),
        scratch_shapes=[
            pltpu.VMEM((NSEG, 16), jnp.float32),
            pltpu.VMEM((NSEG, 1), jnp.int32),
            pltpu.VMEM((NSEG, 1), jnp.int32),
            pltpu.VMEM((32, 1), jnp.float32),
            pltpu.VMEM((NSEG, 1), jnp.float32),
        ],
    )
    out = pl.pallas_call(
        _fused_kernel,
        grid_spec=grid_spec,
        out_shape=jax.ShapeDtypeStruct((NSEG, DIM), jnp.float32),
        compiler_params=pltpu.CompilerParams(
            dimension_semantics=("arbitrary",),
        ),
    )(obs_ptr, b1, b2, h_dag, W1, W2, W3, b3.reshape(1, DIM))
    return out


# speedup vs baseline: 1.2940x; 1.0009x over previous
import jax
import jax.numpy as jnp
from jax.experimental import pallas as pl
from jax.experimental.pallas import tpu as pltpu

N_TOK = 32768
DIM = 128
NSEG = 16
TILE = 8192
GRID = N_TOK // TILE


def _leaky(x):
    return jnp.maximum(x, 0.2 * x)


def _smem_to_col(ref, n, offset=0):
    sub = jax.lax.broadcasted_iota(jnp.int32, (n, 1), 0)
    col = jnp.zeros((n, 1), ref.dtype)
    for s in range(n):
        col = jnp.where(sub == s, ref[s + offset], col)
    return col


def _fused_kernel(ptr_ref, b1_ref, b2_ref, x_ref, w1_ref, w2_ref,
                  w3_ref, b3_ref, out_ref, acc_ref, lo_ref, hi_ref,
                  b1c_ref, b2c_ref):
    pid = pl.program_id(0)

    @pl.when(pid == 0)
    def _init():
        acc_ref[...] = jnp.zeros_like(acc_ref)
        lo_ref[...] = _smem_to_col(ptr_ref, NSEG)
        hi_ref[...] = _smem_to_col(ptr_ref, NSEG, offset=1)
        b1c_ref[...] = _smem_to_col(b1_ref, 32)
        b2c_ref[...] = _smem_to_col(b2_ref, NSEG)

    h1t = _leaky(
        jax.lax.dot_general(w1_ref[...].astype(jnp.bfloat16),
                            x_ref[...].astype(jnp.bfloat16),
                            (((0,), (1,)), ((), ())),
                            preferred_element_type=jnp.float32)
        + b1c_ref[...])
    h2t = _leaky(
        jax.lax.dot_general(w2_ref[...], h1t,
                            (((0,), (0,)), ((), ())),
                            preferred_element_type=jnp.float32)
        + b2c_ref[...])

    cols = jax.lax.broadcasted_iota(jnp.int32, (NSEG, TILE), 1) + pid * TILE
    m = jnp.logical_and(cols >= lo_ref[...], cols < hi_ref[...])

    acc_ref[...] += jax.lax.dot_general(
        m.astype(jnp.bfloat16), h2t.astype(jnp.bfloat16),
        (((1,), (1,)), ((), ())),
        preferred_element_type=jnp.float32)

    @pl.when(pid == GRID - 1)
    def _finish():
        cnt = (hi_ref[...] - lo_ref[...]).astype(jnp.float32)
        out_ref[...] = (
            jnp.dot(acc_ref[...], w3_ref[...], preferred_element_type=jnp.float32)
            + cnt * b3_ref[...]
        )


def kernel(h_dag, obs_ptr, W1, b1, W2, b2, W3, b3):
    const = lambda i, *refs: (0, 0)
    grid_spec = pltpu.PrefetchScalarGridSpec(
        num_scalar_prefetch=3,
        grid=(GRID,),
        in_specs=[
            pl.BlockSpec((TILE, DIM), lambda i, *refs: (i, 0)),
            pl.BlockSpec((DIM, 32), const),
            pl.BlockSpec((32, 16), const),
            pl.BlockSpec((16, DIM), const),
            pl.BlockSpec((1, DIM), const),
        ],
        out_specs=pl.BlockSpec((NSEG, DIM), const),
        scratch_shapes=[
            pltpu.VMEM((NSEG, 16), jnp.float32),
            pltpu.VMEM((NSEG, 1), jnp.int32),
            pltpu.VMEM((NSEG, 1), jnp.int32),
            pltpu.VMEM((32, 1), jnp.float32),
            pltpu.VMEM((NSEG, 1), jnp.float32),
        ],
    )
    out = pl.pallas_call(
        _fused_kernel,
        grid_spec=grid_spec,
        out_shape=jax.ShapeDtypeStruct((NSEG, DIM), jnp.float32),
        compiler_params=pltpu.CompilerParams(
            dimension_semantics=("arbitrary",),
        ),
    )(obs_ptr, b1, b2, h_dag, W1, W2, W3, b3.reshape(1, DIM))
    return out
